# baseline (speedup 1.0000x reference)
import functools

import jax
import jax.numpy as jnp
import numpy as np
from jax import lax
from jax.experimental import pallas as pl
from jax.experimental.pallas import tpu as pltpu
from jax.experimental.pallas import tpu_sc as plsc

_NC = 2
_NS = 16
_NW = _NC * _NS

_S = 32
_FP = 32
_LANES = 128


def _patterns(f):
    fq = (f + 3) // 4
    t = np.arange(_S * _FP)
    bl = t // _FP
    ff = t % _FP
    ffc = np.minimum(ff, 4 * fq - 1)
    d = (bl // 8) * (fq * 32) + (ffc // 4) * 32 + (bl % 8) * 4 + (ffc % 4)
    return d.astype(np.int32)


def _sc_gather(table, idxp, dpat, n_dst, rows_chunk):
    n_idx = idxp.shape[0]
    per_w_samples = (n_idx // _FP) // _NW
    n_chunks = per_w_samples // _S
    n_pairs = n_chunks // 2
    dst_per_chunk = rows_chunk
    dst_per_w = n_chunks * dst_per_chunk
    idx_per_chunk = _S * _FP

    mesh = plsc.VectorSubcoreMesh(core_axis_name="c", subcore_axis_name="s")

    @functools.partial(
        pl.kernel,
        mesh=mesh,
        out_type=jax.ShapeDtypeStruct((n_dst, 32), jnp.float32),
        scratch_types=[
            pltpu.VMEM((idx_per_chunk,), jnp.int32),
            pltpu.VMEM((idx_per_chunk,), jnp.int32),
            pltpu.VMEM((idx_per_chunk,), jnp.int32),
            pltpu.VMEM((idx_per_chunk,), jnp.int32),
            pltpu.VMEM((idx_per_chunk, 32), jnp.float32),
            pltpu.VMEM((idx_per_chunk, 32), jnp.float32),
            pltpu.VMEM((idx_per_chunk,), jnp.int32),
            pltpu.SemaphoreType.DMA,
            pltpu.SemaphoreType.DMA,
            pltpu.SemaphoreType.DMA,
            pltpu.SemaphoreType.DMA,
        ],
        compiler_params=pltpu.CompilerParams(use_tc_tiling_on_sc=False),
    )
    def gather_kernel(table_hbm, idxp_hbm, dpat_hbm, out_hbm,
                      idx0, idx1, dl0, dl1, rows0, rows1, dpat_v,
                      sg0, sg1, sw0, sw1):
        wid = lax.axis_index("s") * _NC + lax.axis_index("c")
        idx_base = wid * per_w_samples * _FP
        dst_base = wid * dst_per_w

        pltpu.sync_copy(dpat_hbm, dpat_v)

        def start_chunk(i, idx_v, rows_v, sem):
            b0 = idx_base + i * idx_per_chunk
            pltpu.sync_copy(idxp_hbm.at[pl.ds(b0, idx_per_chunk)], idx_v)
            pltpu.async_copy(table_hbm.at[idx_v], rows_v, sem)

        def drain_gather(rows_v, sem):
            pltpu.make_async_copy(
                table_hbm.at[pl.ds(0, idx_per_chunk)], rows_v, sem).wait()

        def start_write(i, dl_v, rows_v, sem):
            off = dst_base + i * dst_per_chunk
            for k in range(idx_per_chunk // 16):
                dl_v[pl.ds(k * 16, 16)] = dpat_v[pl.ds(k * 16, 16)] + off
            pltpu.async_copy(rows_v, out_hbm.at[dl_v], sem)

        def drain_write(rows_v, sem):
            pltpu.make_async_copy(
                rows_v, out_hbm.at[pl.ds(0, idx_per_chunk)], sem).wait()

        start_chunk(0, idx0, rows0, sg0)

        def body(p, carry):
            a = 2 * p
            start_chunk(a + 1, idx1, rows1, sg1)
            drain_gather(rows0, sg0)
            start_write(a, dl0, rows0, sw0)

            @pl.when(p < n_pairs - 1)
            def _():
                drain_write(rows0, sw0)
                start_chunk(a + 2, idx0, rows0, sg0)

            drain_gather(rows1, sg1)
            start_write(a + 1, dl1, rows1, sw1)
            drain_write(rows1, sw1)
            return carry

        lax.fori_loop(0, n_pairs, body, 0)
        drain_write(rows0, sw0)

    return gather_kernel(table, idxp, dpat)


def _tc_dense(x3, wq, b2, n_out):
    n_groups = x3.shape[0]
    fq = x3.shape[1] // 8
    m = wq.shape[1]
    gblk = 64

    def dense_kernel(x_ref, w_ref, b_ref, o_ref):
        rows = gblk * 8
        acc = jnp.zeros((rows, m), dtype=jnp.float32)
        for j in range(fq):
            xj = x_ref[:, 8 * j:8 * (j + 1), :].reshape(rows, _LANES)
            acc = acc + jnp.dot(xj, w_ref[128 * j:128 * (j + 1), :],
                                preferred_element_type=jnp.float32)
        o_ref[...] = jnp.maximum(acc + b_ref[...], 0.0)

    return pl.pallas_call(
        dense_kernel,
        grid=(n_groups // gblk,),
        in_specs=[
            pl.BlockSpec((gblk, 8 * fq, _LANES), lambda i: (i, 0, 0)),
            pl.BlockSpec((128 * fq, m), lambda i: (0, 0)),
            pl.BlockSpec((1, m), lambda i: (0, 0)),
        ],
        out_specs=pl.BlockSpec((gblk * 8, m), lambda i: (i, 0)),
        out_shape=jax.ShapeDtypeStruct((n_out, m), jnp.float32),
    )(x3, wq, b2)


def kernel(input, table, W, b):
    bsz, f = input.shape
    fq = (f + 3) // 4
    idx32 = input.astype(jnp.int32)
    idxp = jnp.pad(idx32, ((0, 0), (0, _FP - f))).reshape(-1)
    dpat = _patterns(f)
    rows_chunk = (_S // 8) * fq * 32
    n_dst = (bsz // 8) * fq * 32
    X = _sc_gather(table, idxp, jnp.asarray(dpat), n_dst, rows_chunk)
    x3 = X.reshape(bsz // 8, 8 * fq, _LANES)
    wq = jnp.pad(W, ((0, 128 * fq - W.shape[0]), (0, 0)))
    return _tc_dense(x3, wq, b.reshape(1, -1), bsz)


# speedup vs baseline: 1.8473x; 1.8473x over previous
import functools

import jax
import jax.numpy as jnp
from jax import lax
from jax.experimental import pallas as pl
from jax.experimental.pallas import tpu as pltpu
from jax.experimental.pallas import tpu_sc as plsc

_NC = 2
_NS = 16
_NW = _NC * _NS

_FP = 32
_FQ = 7
_CHUNK = 1024
_LANES = 128


def _sc_gather(table, idxq, n_idx_used):
    d = table.shape[1]
    n_out = idxq.shape[0]
    n_chunks = n_idx_used // _CHUNK
    per_w = n_chunks // _NW
    n_pairs = per_w // 2

    mesh = plsc.VectorSubcoreMesh(core_axis_name="c", subcore_axis_name="s")

    @functools.partial(
        pl.kernel,
        mesh=mesh,
        out_type=jax.ShapeDtypeStruct((n_out, d), jnp.float32),
        scratch_types=[
            pltpu.VMEM((_CHUNK,), jnp.int32),
            pltpu.VMEM((_CHUNK,), jnp.int32),
            pltpu.VMEM((_CHUNK, d), jnp.float32),
            pltpu.VMEM((_CHUNK, d), jnp.float32),
            pltpu.SemaphoreType.DMA,
            pltpu.SemaphoreType.DMA,
            pltpu.SemaphoreType.DMA,
            pltpu.SemaphoreType.DMA,
        ],
        compiler_params=pltpu.CompilerParams(use_tc_tiling_on_sc=False),
    )
    def gather_kernel(table_hbm, idx_hbm, out_hbm,
                      idx0, idx1, rows0, rows1, sg0, sg1, sw0, sw1):
        wid = lax.axis_index("s") * _NC + lax.axis_index("c")
        base = wid * per_w * _CHUNK

        def start_gather(i, idx_v, rows_v, sem):
            off = base + i * _CHUNK
            pltpu.sync_copy(idx_hbm.at[pl.ds(off, _CHUNK)], idx_v)
            pltpu.async_copy(table_hbm.at[idx_v], rows_v, sem)

        def drain_gather(rows_v, sem):
            pltpu.make_async_copy(
                table_hbm.at[pl.ds(0, _CHUNK)], rows_v, sem).wait()

        def start_write(i, rows_v, sem):
            off = base + i * _CHUNK
            pltpu.async_copy(rows_v, out_hbm.at[pl.ds(off, _CHUNK)], sem)

        def drain_write(rows_v, sem):
            pltpu.make_async_copy(
                rows_v, out_hbm.at[pl.ds(base, _CHUNK)], sem).wait()

        start_gather(0, idx0, rows0, sg0)

        def body(j, carry):
            a = 2 * j
            start_gather(a + 1, idx1, rows1, sg1)
            drain_gather(rows0, sg0)
            start_write(a, rows0, sw0)

            @pl.when(j < n_pairs - 1)
            def _():
                drain_write(rows0, sw0)
                start_gather(a + 2, idx0, rows0, sg0)

            drain_gather(rows1, sg1)
            start_write(a + 1, rows1, sw1)
            drain_write(rows1, sw1)
            return carry

        lax.fori_loop(0, n_pairs, body, 0)
        drain_write(rows0, sw0)

    return gather_kernel(table, idxq)


def _tc_dense(x3, wq, b2, n_out):
    bsz = x3.shape[1]
    m = wq.shape[1]
    blk = 1024

    def dense_kernel(x_ref, w_ref, b_ref, o_ref):
        acc = None
        for j in range(_FQ):
            d = jnp.dot(x_ref[j], w_ref[128 * j:128 * (j + 1), :],
                        preferred_element_type=jnp.float32)
            acc = d if acc is None else acc + d
        o_ref[...] = jnp.maximum(acc + b_ref[...], 0.0)

    return pl.pallas_call(
        dense_kernel,
        grid=(bsz // blk,),
        in_specs=[
            pl.BlockSpec((_FQ, blk, _LANES), lambda i: (0, i, 0)),
            pl.BlockSpec((wq.shape[0], m), lambda i: (0, 0)),
            pl.BlockSpec((1, m), lambda i: (0, 0)),
        ],
        out_specs=pl.BlockSpec((blk, m), lambda i: (i, 0)),
        out_shape=jax.ShapeDtypeStruct((n_out, m), jnp.float32),
    )(x3, wq, b2)


def kernel(input, table, W, b):
    bsz, f = input.shape
    idx32 = input.astype(jnp.int32)
    idxq = (jnp.pad(idx32, ((0, 0), (0, _FP - f)))
            .reshape(bsz, _FP // 4, 4)
            .transpose(1, 0, 2)
            .reshape(-1))
    n_idx_used = _FQ * bsz * 4
    X = _sc_gather(table, idxq, n_idx_used)
    x3 = X.reshape(_FP // 4, bsz, _LANES)
    wq = jnp.pad(W, ((0, _FQ * _LANES - W.shape[0]), (0, 0)))
    return _tc_dense(x3, wq, b.reshape(1, -1), bsz)


# speedup vs baseline: 3.3988x; 1.8399x over previous
import functools

import jax
import jax.numpy as jnp
from jax import lax
from jax.experimental import pallas as pl
from jax.experimental.pallas import tpu as pltpu
from jax.experimental.pallas import tpu_sc as plsc

_NC = 2
_NS = 16
_NW = _NC * _NS

_CHUNK = 832


def _sc_gather(table, idx):
    n = idx.shape[0]
    d = table.shape[1]
    per_w = n // _NW
    n_chunks = per_w // _CHUNK
    n_pairs = n_chunks // 2

    mesh = plsc.VectorSubcoreMesh(core_axis_name="c", subcore_axis_name="s")

    @functools.partial(
        pl.kernel,
        mesh=mesh,
        out_type=jax.ShapeDtypeStruct((n, d), jnp.float32),
        scratch_types=[
            pltpu.VMEM((_CHUNK,), jnp.int32),
            pltpu.VMEM((_CHUNK,), jnp.int32),
            pltpu.VMEM((_CHUNK, d), jnp.float32),
            pltpu.VMEM((_CHUNK, d), jnp.float32),
            pltpu.SemaphoreType.DMA,
            pltpu.SemaphoreType.DMA,
            pltpu.SemaphoreType.DMA,
            pltpu.SemaphoreType.DMA,
        ],
        compiler_params=pltpu.CompilerParams(use_tc_tiling_on_sc=False),
    )
    def gather_kernel(table_hbm, idx_hbm, out_hbm,
                      idx0, idx1, rows0, rows1, sg0, sg1, sw0, sw1):
        wid = lax.axis_index("s") * _NC + lax.axis_index("c")
        base = wid * per_w

        def start_gather(i, idx_v, rows_v, sem):
            off = base + i * _CHUNK
            pltpu.sync_copy(idx_hbm.at[pl.ds(off, _CHUNK)], idx_v)
            pltpu.async_copy(table_hbm.at[idx_v], rows_v, sem)

        def drain_gather(rows_v, sem):
            pltpu.make_async_copy(table_hbm.at[pl.ds(0, _CHUNK)], rows_v, sem).wait()

        def start_write(i, rows_v, sem):
            pltpu.async_copy(rows_v, out_hbm.at[pl.ds(base + i * _CHUNK, _CHUNK)], sem)

        def drain_write(rows_v, sem):
            pltpu.make_async_copy(rows_v, out_hbm.at[pl.ds(base, _CHUNK)], sem).wait()

        start_gather(0, idx0, rows0, sg0)

        def body(j, carry):
            a = 2 * j
            start_gather(a + 1, idx1, rows1, sg1)
            drain_gather(rows0, sg0)
            start_write(a, rows0, sw0)
            @pl.when(j < n_pairs - 1)
            def _():
                drain_write(rows0, sw0)
                start_gather(a + 2, idx0, rows0, sg0)
            drain_gather(rows1, sg1)
            start_write(a + 1, rows1, sw1)
            drain_write(rows1, sw1)
            return carry

        lax.fori_loop(0, n_pairs, body, 0)
        drain_write(rows0, sw0)

    return gather_kernel(table, idx)


def _tc_dense(x, w, b):
    bsz, k = x.shape
    m = w.shape[1]
    bm = 1024

    def dense_kernel(x_ref, w_ref, b_ref, o_ref):
        acc = jnp.dot(x_ref[...], w_ref[...], preferred_element_type=jnp.float32)
        o_ref[...] = jnp.maximum(acc + b_ref[...], 0.0)

    return pl.pallas_call(
        dense_kernel,
        grid=(bsz // bm,),
        in_specs=[
            pl.BlockSpec((bm, k), lambda i: (i, 0)),
            pl.BlockSpec((k, m), lambda i: (0, 0)),
            pl.BlockSpec((1, m), lambda i: (0, 0)),
        ],
        out_specs=pl.BlockSpec((bm, m), lambda i: (i, 0)),
        out_shape=jax.ShapeDtypeStruct((bsz, m), jnp.float32),
    )(x, w, b)


def kernel(input, table, W, b):
    bsz, f = input.shape
    d = table.shape[1]
    idx = input.reshape(-1).astype(jnp.int32)
    gathered = _sc_gather(table, idx)
    x = gathered.reshape(bsz, f * d)
    return _tc_dense(x, W, b.reshape(1, -1))


# speedup vs baseline: 3.4382x; 1.0116x over previous
import functools

import jax
import jax.numpy as jnp
from jax import lax
from jax.experimental import pallas as pl
from jax.experimental.pallas import tpu as pltpu
from jax.experimental.pallas import tpu_sc as plsc

_NC = 2
_NS = 16
_NW = _NC * _NS

_CHUNK = 1664


def _sc_gather(table, idx):
    n = idx.shape[0]
    d = table.shape[1]
    per_w = n // _NW
    n_chunks = per_w // _CHUNK
    n_pairs = n_chunks // 2

    mesh = plsc.VectorSubcoreMesh(core_axis_name="c", subcore_axis_name="s")

    @functools.partial(
        pl.kernel,
        mesh=mesh,
        out_type=jax.ShapeDtypeStruct((n, d), jnp.float32),
        scratch_types=[
            pltpu.VMEM((_CHUNK,), jnp.int32),
            pltpu.VMEM((_CHUNK,), jnp.int32),
            pltpu.VMEM((_CHUNK, d), jnp.float32),
            pltpu.VMEM((_CHUNK, d), jnp.float32),
            pltpu.SemaphoreType.DMA,
            pltpu.SemaphoreType.DMA,
            pltpu.SemaphoreType.DMA,
            pltpu.SemaphoreType.DMA,
        ],
        compiler_params=pltpu.CompilerParams(use_tc_tiling_on_sc=False),
    )
    def gather_kernel(table_hbm, idx_hbm, out_hbm,
                      idx0, idx1, rows0, rows1, sg0, sg1, sw0, sw1):
        wid = lax.axis_index("s") * _NC + lax.axis_index("c")
        base = wid * per_w

        def start_gather(i, idx_v, rows_v, sem):
            off = base + i * _CHUNK
            pltpu.sync_copy(idx_hbm.at[pl.ds(off, _CHUNK)], idx_v)
            pltpu.async_copy(table_hbm.at[idx_v], rows_v, sem)

        def drain_gather(rows_v, sem):
            pltpu.make_async_copy(table_hbm.at[pl.ds(0, _CHUNK)], rows_v, sem).wait()

        def start_write(i, rows_v, sem):
            pltpu.async_copy(rows_v, out_hbm.at[pl.ds(base + i * _CHUNK, _CHUNK)], sem)

        def drain_write(rows_v, sem):
            pltpu.make_async_copy(rows_v, out_hbm.at[pl.ds(base, _CHUNK)], sem).wait()

        start_gather(0, idx0, rows0, sg0)

        def body(j, carry):
            a = 2 * j
            start_gather(a + 1, idx1, rows1, sg1)
            drain_gather(rows0, sg0)
            start_write(a, rows0, sw0)
            @pl.when(j < n_pairs - 1)
            def _():
                drain_write(rows0, sw0)
                start_gather(a + 2, idx0, rows0, sg0)
            drain_gather(rows1, sg1)
            start_write(a + 1, rows1, sw1)
            drain_write(rows1, sw1)
            return carry

        lax.fori_loop(0, n_pairs, body, 0)
        drain_write(rows0, sw0)

    return gather_kernel(table, idx)


def _tc_dense(x, w, b):
    bsz, k = x.shape
    m = w.shape[1]
    bm = 1024

    def dense_kernel(x_ref, w_ref, b_ref, o_ref):
        acc = jnp.dot(x_ref[...], w_ref[...], preferred_element_type=jnp.float32)
        o_ref[...] = jnp.maximum(acc + b_ref[...], 0.0)

    return pl.pallas_call(
        dense_kernel,
        grid=(bsz // bm,),
        in_specs=[
            pl.BlockSpec((bm, k), lambda i: (i, 0)),
            pl.BlockSpec((k, m), lambda i: (0, 0)),
            pl.BlockSpec((1, m), lambda i: (0, 0)),
        ],
        out_specs=pl.BlockSpec((bm, m), lambda i: (i, 0)),
        out_shape=jax.ShapeDtypeStruct((bsz, m), jnp.float32),
    )(x, w, b)


def kernel(input, table, W, b):
    bsz, f = input.shape
    d = table.shape[1]
    idx = input.reshape(-1).astype(jnp.int32)
    gathered = _sc_gather(table, idx)
    x = gathered.reshape(bsz, f * d)
    return _tc_dense(x, W, b.reshape(1, -1))
